# baseline (speedup 1.0000x reference)
import functools
import math

import jax
import jax.numpy as jnp
from jax import lax
from jax.experimental import pallas as pl
from jax.experimental.pallas import tpu as pltpu
from jax.experimental.pallas import tpu_sc as plsc

PAD = 0
V = 32000
EPS = 0.1 / (V - 2)
CONF = 0.9
C_CONST = 0.1 * math.log(EPS) + CONF * math.log(CONF)

NC, NS, LANES = 2, 16, 16
NW = NC * NS
N_ROWS = 2048
ROWS_PER_TILE = N_ROWS // NW
ROW_WORDS = V // 128

W_SC = 3200
V_TC = V - W_SC
UNROLL = 10

K_STREAMS = 1
BR, BC = 256, 3200
GR = N_ROWS // BR
GC = V_TC // (BC * K_STREAMS)


def _sc_gather_body(inp_hbm, lab_hbm, out_hbm,
                    lab_v, idx_t, idx_0, val_t, val_0, out_v,
                    buf0, buf1, sem, sem0, sem1):
    c = lax.axis_index("c")
    s = lax.axis_index("s")
    wid = s * NC + c
    base = wid * ROWS_PER_TILE
    pltpu.sync_copy(lab_hbm.at[pl.ds(base, ROWS_PER_TILE)], lab_v)
    for k in range(ROWS_PER_TILE // LANES):
        lab = lab_v[pl.ds(k * LANES, LANES)]
        row = base + k * LANES + lax.iota(jnp.int32, LANES)
        idx_t[pl.ds(k * LANES, LANES)] = row * V + lab
        idx_0[pl.ds(k * LANES, LANES)] = row * V
    cp_t = pltpu.async_copy(inp_hbm.at[idx_t], val_t, sem)
    cp_0 = pltpu.async_copy(inp_hbm.at[idx_0], val_0, sem)
    cp_t.wait()
    cp_0.wait()
    acc = jnp.zeros((LANES,), jnp.float32)
    for k in range(ROWS_PER_TILE // LANES):
        lab = lab_v[pl.ds(k * LANES, LANES)]
        xt = val_t[pl.ds(k * LANES, LANES)]
        x0 = val_0[pl.ds(k * LANES, LANES)]
        contrib = C_CONST + EPS * x0 - (CONF - EPS) * xt
        acc = acc + jnp.where(lab != PAD, contrib, 0.0)

    bufs = (buf0, buf1)
    sems = (sem0, sem1)
    cps = [None, None]

    def _start(j):
        b = j % 2
        cps[b] = pltpu.async_copy(
            inp_hbm.at[pl.ds((base + j) * V + V_TC, W_SC)], bufs[b], sems[b])

    _start(0)
    for k in range(128 // LANES):
        out_v[pl.ds(k * LANES, LANES)] = jnp.zeros((LANES,), jnp.float32)
    for j in range(ROWS_PER_TILE):
        if j + 1 < ROWS_PER_TILE:
            _start(j + 1)
        cps[j % 2].wait()
        lab_j = lab_v[pl.ds((j // LANES) * LANES, LANES)][j % LANES]

        @pl.when(lab_j != PAD)
        def _(j=j):
            bufb = bufs[j % 2]

            def body(i, t):
                for k in range(UNROLL):
                    t = t + bufb[pl.ds(i * (LANES * UNROLL) + LANES * k,
                                       LANES)]
                return t

            srow = lax.fori_loop(0, W_SC // (LANES * UNROLL), body,
                                 jnp.zeros((LANES,), jnp.float32))
            out_v[pl.ds(LANES, LANES)] = out_v[pl.ds(LANES, LANES)] + srow

    sum_acc = out_v[pl.ds(LANES, LANES)]
    out_v[pl.ds(LANES, LANES)] = -EPS * sum_acc
    out_v[pl.ds(0, LANES)] = acc
    pltpu.sync_copy(out_v, out_hbm.at[wid])


def _tc_body(*refs):
    x_refs = refs[:K_STREAMS]
    lab_ref, out_ref = refs[K_STREAMS:]
    r = pl.program_id(0)
    c = pl.program_id(1)
    partial = jnp.sum(x_refs[0][...], axis=1)
    for xr in x_refs[1:]:
        partial = partial + jnp.sum(xr[...], axis=1)
    lab = lab_ref[0, 0, :]
    masked = jnp.where(lab != PAD, partial, 0.0)
    val = -EPS * jnp.sum(masked)

    @pl.when((r == 0) & (c == 0))
    def _init():
        out_ref[0, 0] = 0.0

    out_ref[0, 0] += val


def kernel(inputs, labels):
    lab_flat = labels.reshape(-1).astype(jnp.int32)
    x_flat = inputs.reshape(N_ROWS * V)

    sc_call = functools.partial(
        pl.kernel,
        mesh=plsc.VectorSubcoreMesh(core_axis_name="c", subcore_axis_name="s"),
        out_type=jax.ShapeDtypeStruct((NW, 128), jnp.float32),
        scratch_types=[
            pltpu.VMEM((ROWS_PER_TILE,), jnp.int32),
            pltpu.VMEM((ROWS_PER_TILE,), jnp.int32),
            pltpu.VMEM((ROWS_PER_TILE,), jnp.int32),
            pltpu.VMEM((ROWS_PER_TILE,), jnp.float32),
            pltpu.VMEM((ROWS_PER_TILE,), jnp.float32),
            pltpu.VMEM((128,), jnp.float32),
            pltpu.VMEM((W_SC,), jnp.float32),
            pltpu.VMEM((W_SC,), jnp.float32),
            pltpu.SemaphoreType.DMA,
            pltpu.SemaphoreType.DMA,
            pltpu.SemaphoreType.DMA,
        ],
    )(_sc_gather_body)
    scp = sc_call(x_flat, lab_flat)

    x2d = inputs.reshape(N_ROWS, V)
    lab3 = lab_flat.reshape(GR, 1, BR)
    x_specs = [
        pl.BlockSpec((BR, BC),
                     lambda r, c, k=k: (r, c * K_STREAMS + k))
        for k in range(K_STREAMS)
    ]
    out = pl.pallas_call(
        _tc_body,
        grid=(GR, GC),
        in_specs=x_specs + [
            pl.BlockSpec((1, 1, BR), lambda r, c: (r, 0, 0)),
        ],
        out_specs=pl.BlockSpec((1, 1), lambda r, c: (0, 0),
                               memory_space=pltpu.SMEM),
        out_shape=jax.ShapeDtypeStruct((1, 1), jnp.float32),
        compiler_params=pltpu.CompilerParams(
            dimension_semantics=("arbitrary", "arbitrary")),
    )(*([x2d] * K_STREAMS), lab3)
    return out[0, 0] + jnp.sum(scp)


# speedup vs baseline: 1.0635x; 1.0635x over previous
import functools
import math

import jax
import jax.numpy as jnp
from jax import lax
from jax.experimental import pallas as pl
from jax.experimental.pallas import tpu as pltpu
from jax.experimental.pallas import tpu_sc as plsc

PAD = 0
V = 32000
EPS = 0.1 / (V - 2)
CONF = 0.9
C_CONST = 0.1 * math.log(EPS) + CONF * math.log(CONF)

NC, NS, LANES = 2, 16, 16
NW = NC * NS
N_ROWS = 2048
ROWS_PER_TILE = N_ROWS // NW
ROW_WORDS = V // 128

W_SC = 3200
V_TC = V - W_SC
UNROLL = 10

K_STREAMS = 1
BR, BC = 256, 9600
GR = N_ROWS // BR
GC = V_TC // (BC * K_STREAMS)


def _sc_gather_body(inp_hbm, lab_hbm, out_hbm,
                    lab_v, idx_t, idx_0, val_t, val_0, out_v,
                    buf0, buf1, sem, sem0, sem1):
    c = lax.axis_index("c")
    s = lax.axis_index("s")
    wid = s * NC + c
    base = wid * ROWS_PER_TILE
    pltpu.sync_copy(lab_hbm.at[pl.ds(base, ROWS_PER_TILE)], lab_v)
    for k in range(ROWS_PER_TILE // LANES):
        lab = lab_v[pl.ds(k * LANES, LANES)]
        row = base + k * LANES + lax.iota(jnp.int32, LANES)
        idx_t[pl.ds(k * LANES, LANES)] = row * V + lab
        idx_0[pl.ds(k * LANES, LANES)] = row * V
    cp_t = pltpu.async_copy(inp_hbm.at[idx_t], val_t, sem)
    cp_0 = pltpu.async_copy(inp_hbm.at[idx_0], val_0, sem)
    cp_t.wait()
    cp_0.wait()
    acc = jnp.zeros((LANES,), jnp.float32)
    for k in range(ROWS_PER_TILE // LANES):
        lab = lab_v[pl.ds(k * LANES, LANES)]
        xt = val_t[pl.ds(k * LANES, LANES)]
        x0 = val_0[pl.ds(k * LANES, LANES)]
        contrib = C_CONST + EPS * x0 - (CONF - EPS) * xt
        acc = acc + jnp.where(lab != PAD, contrib, 0.0)

    bufs = (buf0, buf1)
    sems = (sem0, sem1)
    cps = [None, None]

    def _start(j):
        b = j % 2
        cps[b] = pltpu.async_copy(
            inp_hbm.at[pl.ds((base + j) * V + V_TC, W_SC)], bufs[b], sems[b])

    _start(0)
    for k in range(128 // LANES):
        out_v[pl.ds(k * LANES, LANES)] = jnp.zeros((LANES,), jnp.float32)
    for j in range(ROWS_PER_TILE):
        if j + 1 < ROWS_PER_TILE:
            _start(j + 1)
        cps[j % 2].wait()
        lab_j = lab_v[pl.ds((j // LANES) * LANES, LANES)][j % LANES]

        @pl.when(lab_j != PAD)
        def _(j=j):
            bufb = bufs[j % 2]

            def body(i, t):
                for k in range(UNROLL):
                    t = t + bufb[pl.ds(i * (LANES * UNROLL) + LANES * k,
                                       LANES)]
                return t

            srow = lax.fori_loop(0, W_SC // (LANES * UNROLL), body,
                                 jnp.zeros((LANES,), jnp.float32))
            out_v[pl.ds(LANES, LANES)] = out_v[pl.ds(LANES, LANES)] + srow

    sum_acc = out_v[pl.ds(LANES, LANES)]
    out_v[pl.ds(LANES, LANES)] = -EPS * sum_acc
    out_v[pl.ds(0, LANES)] = acc
    pltpu.sync_copy(out_v, out_hbm.at[wid])


def _tc_body(*refs):
    x_refs = refs[:K_STREAMS]
    lab_ref, out_ref = refs[K_STREAMS:]
    r = pl.program_id(0)
    c = pl.program_id(1)
    partial = jnp.sum(x_refs[0][...], axis=1)
    for xr in x_refs[1:]:
        partial = partial + jnp.sum(xr[...], axis=1)
    lab = lab_ref[0, 0, :]
    masked = jnp.where(lab != PAD, partial, 0.0)
    val = -EPS * jnp.sum(masked)

    @pl.when((r == 0) & (c == 0))
    def _init():
        out_ref[0, 0] = 0.0

    out_ref[0, 0] += val


def kernel(inputs, labels):
    lab_flat = labels.reshape(-1).astype(jnp.int32)
    x_flat = inputs.reshape(N_ROWS * V)

    sc_call = functools.partial(
        pl.kernel,
        mesh=plsc.VectorSubcoreMesh(core_axis_name="c", subcore_axis_name="s"),
        out_type=jax.ShapeDtypeStruct((NW, 128), jnp.float32),
        scratch_types=[
            pltpu.VMEM((ROWS_PER_TILE,), jnp.int32),
            pltpu.VMEM((ROWS_PER_TILE,), jnp.int32),
            pltpu.VMEM((ROWS_PER_TILE,), jnp.int32),
            pltpu.VMEM((ROWS_PER_TILE,), jnp.float32),
            pltpu.VMEM((ROWS_PER_TILE,), jnp.float32),
            pltpu.VMEM((128,), jnp.float32),
            pltpu.VMEM((W_SC,), jnp.float32),
            pltpu.VMEM((W_SC,), jnp.float32),
            pltpu.SemaphoreType.DMA,
            pltpu.SemaphoreType.DMA,
            pltpu.SemaphoreType.DMA,
        ],
    )(_sc_gather_body)
    scp = sc_call(x_flat, lab_flat)

    x2d = inputs.reshape(N_ROWS, V)
    lab3 = lab_flat.reshape(GR, 1, BR)
    x_specs = [
        pl.BlockSpec((BR, BC),
                     lambda r, c, k=k: (r, c * K_STREAMS + k))
        for k in range(K_STREAMS)
    ]
    out = pl.pallas_call(
        _tc_body,
        grid=(GR, GC),
        in_specs=x_specs + [
            pl.BlockSpec((1, 1, BR), lambda r, c: (r, 0, 0)),
        ],
        out_specs=pl.BlockSpec((1, 1), lambda r, c: (0, 0),
                               memory_space=pltpu.SMEM),
        out_shape=jax.ShapeDtypeStruct((1, 1), jnp.float32),
        compiler_params=pltpu.CompilerParams(
            dimension_semantics=("arbitrary", "arbitrary")),
    )(*([x2d] * K_STREAMS), lab3)
    return out[0, 0] + jnp.sum(scp)


# speedup vs baseline: 1.0739x; 1.0097x over previous
import functools
import math

import jax
import jax.numpy as jnp
from jax import lax
from jax.experimental import pallas as pl
from jax.experimental.pallas import tpu as pltpu
from jax.experimental.pallas import tpu_sc as plsc

PAD = 0
V = 32000
EPS = 0.1 / (V - 2)
CONF = 0.9
C_CONST = 0.1 * math.log(EPS) + CONF * math.log(CONF)

NC, NS, LANES = 2, 16, 16
NW = NC * NS
N_ROWS = 2048
ROWS_PER_TILE = N_ROWS // NW

BR, BC = 256, 16000
GR, GC = N_ROWS // BR, V // BC


def _sc_gather_body(inp_hbm, lab_hbm, out_hbm,
                    lab_v, idx_t, idx_0, val_t, val_0, out_v, sem):
    c = lax.axis_index("c")
    s = lax.axis_index("s")
    wid = s * NC + c
    base = wid * ROWS_PER_TILE
    pltpu.sync_copy(lab_hbm.at[pl.ds(base, ROWS_PER_TILE)], lab_v)
    for k in range(ROWS_PER_TILE // LANES):
        lab = lab_v[pl.ds(k * LANES, LANES)]
        row = base + k * LANES + lax.iota(jnp.int32, LANES)
        idx_t[pl.ds(k * LANES, LANES)] = row * V + lab
        idx_0[pl.ds(k * LANES, LANES)] = row * V
    cp_t = pltpu.async_copy(inp_hbm.at[idx_t], val_t, sem)
    cp_0 = pltpu.async_copy(inp_hbm.at[idx_0], val_0, sem)
    cp_t.wait()
    cp_0.wait()
    acc = jnp.zeros((LANES,), jnp.float32)
    for k in range(ROWS_PER_TILE // LANES):
        lab = lab_v[pl.ds(k * LANES, LANES)]
        xt = val_t[pl.ds(k * LANES, LANES)]
        x0 = val_0[pl.ds(k * LANES, LANES)]
        contrib = C_CONST + EPS * x0 - (CONF - EPS) * xt
        acc = acc + jnp.where(lab != PAD, contrib, 0.0)
    for k in range(128 // LANES):
        out_v[pl.ds(k * LANES, LANES)] = jnp.zeros((LANES,), jnp.float32)
    out_v[pl.ds(0, LANES)] = acc
    pltpu.sync_copy(out_v, out_hbm.at[wid])


def _tc_body(x_ref, lab_ref, scp_ref, out_ref):
    r = pl.program_id(0)
    c = pl.program_id(1)
    partial = jnp.sum(x_ref[...], axis=1)
    lab = lab_ref[0, 0, :]
    masked = jnp.where(lab != PAD, partial, 0.0)
    val = -EPS * jnp.sum(masked)

    @pl.when((r == 0) & (c == 0))
    def _init():
        out_ref[0, 0] = jnp.sum(scp_ref[...])

    out_ref[0, 0] += val


def kernel(inputs, labels):
    lab_flat = labels.reshape(-1).astype(jnp.int32)
    x_flat = inputs.reshape(N_ROWS * V)

    sc_call = functools.partial(
        pl.kernel,
        mesh=plsc.VectorSubcoreMesh(core_axis_name="c", subcore_axis_name="s"),
        out_type=jax.ShapeDtypeStruct((NW, 128), jnp.float32),
        scratch_types=[
            pltpu.VMEM((ROWS_PER_TILE,), jnp.int32),
            pltpu.VMEM((ROWS_PER_TILE,), jnp.int32),
            pltpu.VMEM((ROWS_PER_TILE,), jnp.int32),
            pltpu.VMEM((ROWS_PER_TILE,), jnp.float32),
            pltpu.VMEM((ROWS_PER_TILE,), jnp.float32),
            pltpu.VMEM((128,), jnp.float32),
            pltpu.SemaphoreType.DMA,
        ],
    )(_sc_gather_body)
    scp = sc_call(x_flat, lab_flat)

    x2d = inputs.reshape(N_ROWS, V)
    lab3 = lab_flat.reshape(GR, 1, BR)
    out = pl.pallas_call(
        _tc_body,
        grid=(GR, GC),
        in_specs=[
            pl.BlockSpec((BR, BC), lambda r, c: (r, c)),
            pl.BlockSpec((1, 1, BR), lambda r, c: (r, 0, 0)),
            pl.BlockSpec((NW, 128), lambda r, c: (0, 0)),
        ],
        out_specs=pl.BlockSpec((1, 1), lambda r, c: (0, 0),
                               memory_space=pltpu.SMEM),
        out_shape=jax.ShapeDtypeStruct((1, 1), jnp.float32),
        compiler_params=pltpu.CompilerParams(
            dimension_semantics=("arbitrary", "arbitrary")),
    )(x2d, lab3, scp)
    return out[0, 0]


# speedup vs baseline: 1.0768x; 1.0027x over previous
import functools
import math

import jax
import jax.numpy as jnp
from jax import lax
from jax.experimental import pallas as pl
from jax.experimental.pallas import tpu as pltpu
from jax.experimental.pallas import tpu_sc as plsc

PAD = 0
V = 32000
EPS = 0.1 / (V - 2)
CONF = 0.9
C_CONST = 0.1 * math.log(EPS) + CONF * math.log(CONF)

NC, NS, LANES = 2, 16, 16
NW = NC * NS
N_ROWS = 2048
ROWS_PER_TILE = N_ROWS // NW

BR, BC = 128, 32000
GR, GC = N_ROWS // BR, V // BC


def _sc_gather_body(inp_hbm, lab_hbm, out_hbm,
                    lab_v, idx_t, idx_0, val_t, val_0, out_v, sem):
    c = lax.axis_index("c")
    s = lax.axis_index("s")
    wid = s * NC + c
    base = wid * ROWS_PER_TILE
    pltpu.sync_copy(lab_hbm.at[pl.ds(base, ROWS_PER_TILE)], lab_v)
    for k in range(ROWS_PER_TILE // LANES):
        lab = lab_v[pl.ds(k * LANES, LANES)]
        row = base + k * LANES + lax.iota(jnp.int32, LANES)
        idx_t[pl.ds(k * LANES, LANES)] = row * V + lab
        idx_0[pl.ds(k * LANES, LANES)] = row * V
    cp_t = pltpu.async_copy(inp_hbm.at[idx_t], val_t, sem)
    cp_0 = pltpu.async_copy(inp_hbm.at[idx_0], val_0, sem)
    cp_t.wait()
    cp_0.wait()
    acc = jnp.zeros((LANES,), jnp.float32)
    for k in range(ROWS_PER_TILE // LANES):
        lab = lab_v[pl.ds(k * LANES, LANES)]
        xt = val_t[pl.ds(k * LANES, LANES)]
        x0 = val_0[pl.ds(k * LANES, LANES)]
        contrib = C_CONST + EPS * x0 - (CONF - EPS) * xt
        acc = acc + jnp.where(lab != PAD, contrib, 0.0)
    for k in range(128 // LANES):
        out_v[pl.ds(k * LANES, LANES)] = jnp.zeros((LANES,), jnp.float32)
    out_v[pl.ds(0, LANES)] = acc
    pltpu.sync_copy(out_v, out_hbm.at[wid])


def _tc_body(x_ref, lab_ref, scp_ref, out_ref):
    r = pl.program_id(0)
    c = pl.program_id(1)
    partial = jnp.sum(x_ref[...], axis=1)
    lab = lab_ref[0, 0, :]
    masked = jnp.where(lab != PAD, partial, 0.0)
    val = -EPS * jnp.sum(masked)

    @pl.when((r == 0) & (c == 0))
    def _init():
        out_ref[0, 0] = jnp.sum(scp_ref[...])

    out_ref[0, 0] += val


def kernel(inputs, labels):
    lab_flat = labels.reshape(-1).astype(jnp.int32)
    x_flat = inputs.reshape(N_ROWS * V)

    sc_call = functools.partial(
        pl.kernel,
        mesh=plsc.VectorSubcoreMesh(core_axis_name="c", subcore_axis_name="s"),
        out_type=jax.ShapeDtypeStruct((NW, 128), jnp.float32),
        scratch_types=[
            pltpu.VMEM((ROWS_PER_TILE,), jnp.int32),
            pltpu.VMEM((ROWS_PER_TILE,), jnp.int32),
            pltpu.VMEM((ROWS_PER_TILE,), jnp.int32),
            pltpu.VMEM((ROWS_PER_TILE,), jnp.float32),
            pltpu.VMEM((ROWS_PER_TILE,), jnp.float32),
            pltpu.VMEM((128,), jnp.float32),
            pltpu.SemaphoreType.DMA,
        ],
    )(_sc_gather_body)
    scp = sc_call(x_flat, lab_flat)

    x2d = inputs.reshape(N_ROWS, V)
    lab3 = lab_flat.reshape(GR, 1, BR)
    out = pl.pallas_call(
        _tc_body,
        grid=(GR, GC),
        in_specs=[
            pl.BlockSpec((BR, BC), lambda r, c: (r, c)),
            pl.BlockSpec((1, 1, BR), lambda r, c: (r, 0, 0)),
            pl.BlockSpec((NW, 128), lambda r, c: (0, 0)),
        ],
        out_specs=pl.BlockSpec((1, 1), lambda r, c: (0, 0),
                               memory_space=pltpu.SMEM),
        out_shape=jax.ShapeDtypeStruct((1, 1), jnp.float32),
        compiler_params=pltpu.CompilerParams(
            dimension_semantics=("arbitrary", "arbitrary")),
    )(x2d, lab3, scp)
    return out[0, 0]
